# baseline (speedup 1.0000x reference)
import functools

import jax
import jax.numpy as jnp
from jax import lax
from jax.experimental import pallas as pl
from jax.experimental.pallas import tpu as pltpu
from jax.experimental.pallas import tpu_sc as plsc

B = 4 * 8192
D = 1024
V = 8192
P = 8192
NC, NS = 2, 16
NW = NC * NS
SC_PER_W = (B - P) // NW
CHUNK = 32
NCHUNK = SC_PER_W // CHUNK
R = 512
U = 16


def _sc_kernel(table_hbm, idx_hbm, out_hbm, idx_v, buf0, buf1, sem0, sem1):
    wid = lax.axis_index("s") * NC + lax.axis_index("c")
    base = P + wid * SC_PER_W
    pltpu.sync_copy(idx_hbm.at[pl.ds(base, SC_PER_W)], idx_v)

    def gather_cp(g, buf, sem):
        return pltpu.make_async_copy(
            table_hbm.at[idx_v.at[pl.ds(g * CHUNK, CHUNK)]], buf, sem
        )

    def write(g, buf):
        pltpu.sync_copy(buf, out_hbm.at[pl.ds(base + g * CHUNK, CHUNK)])

    gather_cp(0, buf0, sem0).start()

    @pl.loop(0, NCHUNK, step=2)
    def _(g):
        gather_cp(g + 1, buf1, sem1).start()
        gather_cp(g, buf0, sem0).wait()
        write(g, buf0)

        @pl.when(g + 2 < NCHUNK)
        def _():
            gather_cp(g + 2, buf0, sem0).start()

        gather_cp(g + 1, buf1, sem1).wait()
        write(g + 1, buf1)


def _tc_body(idx_ref, table_ref, out_ref):
    i = pl.program_id(0)
    base = i * R
    for r in range(0, R, U):
        vals = [table_ref[idx_ref[base + r + u]] for u in range(U)]
        for u in range(U):
            out_ref[r + u] = vals[u]


def kernel(position_ids, embedding_weight):
    idx = position_ids.reshape(B).astype(jnp.int32)

    mesh = plsc.VectorSubcoreMesh(core_axis_name="c", subcore_axis_name="s")
    sc_gather = functools.partial(
        pl.kernel,
        mesh=mesh,
        out_type=jax.ShapeDtypeStruct((B, D), jnp.float32),
        scratch_types=[
            pltpu.VMEM((SC_PER_W,), jnp.int32),
            pltpu.VMEM((CHUNK, D), jnp.float32),
            pltpu.VMEM((CHUNK, D), jnp.float32),
            pltpu.SemaphoreType.DMA,
            pltpu.SemaphoreType.DMA,
        ],
    )(_sc_kernel)
    sc_out = sc_gather(embedding_weight, idx)

    table3 = embedding_weight.reshape(V, 8, 128)
    grid_spec = pltpu.PrefetchScalarGridSpec(
        num_scalar_prefetch=1,
        grid=(P // R,),
        in_specs=[pl.BlockSpec((V, 8, 128), lambda i, idx_ref: (0, 0, 0))],
        out_specs=pl.BlockSpec((R, 8, 128), lambda i, idx_ref: (i, 0, 0)),
    )
    tc_out = pl.pallas_call(
        _tc_body,
        grid_spec=grid_spec,
        out_shape=jax.ShapeDtypeStruct((P, 8, 128), jnp.float32),
    )(idx, table3)

    out = lax.dynamic_update_slice(sc_out, tc_out.reshape(P, D), (0, 0))
    return out.reshape(4, 8192, D)
